# baseline (speedup 1.0000x reference)
import jax
import jax.numpy as jnp
from jax import lax
from jax.experimental import pallas as pl
from jax.experimental.pallas import tpu as pltpu


def _body(z_ref, gwt_ref, bwt_ref, bg_ref, bb_ref, bm_ref, bv_ref,
          xt_ref, o_ref, params_ref):
    b = pl.program_id(0)
    h = pl.program_id(1)
    B = z_ref.shape[0]
    C = xt_ref.shape[2]

    @pl.when(jnp.logical_and(b == 0, h == 0))
    def _compute_params():
        eps = 1e-3
        a = bg_ref[...] * lax.rsqrt(bv_ref[...] + eps)
        c0 = bb_ref[...] - bm_ref[...] * a
        g = lax.dot_general(z_ref[...], gwt_ref[...],
                            (((1,), (1,)), ((), ())),
                            preferred_element_type=jnp.float32)
        bbm = lax.dot_general(z_ref[...], bwt_ref[...],
                              (((1,), (1,)), ((), ())),
                              preferred_element_type=jnp.float32)
        onepg = 1.0 + g
        params_ref[0:B, :] = a * onepg
        params_ref[B:2 * B, :] = c0 * onepg + bbm

    sr = params_ref[pl.ds(b, 1), :]
    br = params_ref[pl.ds(B + b, 1), :]
    m2 = jnp.concatenate([sr, br], axis=0)
    eye = (lax.broadcasted_iota(jnp.int32, (C, C), 0)
           == lax.broadcasted_iota(jnp.int32, (C, C), 1)
           ).astype(jnp.float32)
    mt = lax.dot_general(eye, m2, (((1,), (1,)), ((), ())),
                         preferred_element_type=jnp.float32)
    s = mt[:, 0:1].reshape(1, 1, C, 1)
    t = mt[:, 1:2].reshape(1, 1, C, 1)
    o_ref[...] = xt_ref[...] * s + t


def kernel(x, z, bn_gamma, bn_beta, bn_mean, bn_var, gamma_w, beta_w):
    B, H, W, C = x.shape
    BH = 28

    xt = jnp.transpose(x, (0, 1, 3, 2))
    gw_t = gamma_w.T
    bw_t = beta_w.T

    out_t = pl.pallas_call(
        _body,
        grid=(B, H // BH),
        in_specs=[
            pl.BlockSpec((B, z.shape[1]), lambda b, h: (0, 0)),
            pl.BlockSpec((C, gw_t.shape[1]), lambda b, h: (0, 0)),
            pl.BlockSpec((C, bw_t.shape[1]), lambda b, h: (0, 0)),
            pl.BlockSpec((1, C), lambda b, h: (0, 0)),
            pl.BlockSpec((1, C), lambda b, h: (0, 0)),
            pl.BlockSpec((1, C), lambda b, h: (0, 0)),
            pl.BlockSpec((1, C), lambda b, h: (0, 0)),
            pl.BlockSpec((1, BH, C, W), lambda b, h: (b, h, 0, 0)),
        ],
        out_specs=pl.BlockSpec((1, BH, C, W), lambda b, h: (b, h, 0, 0)),
        out_shape=jax.ShapeDtypeStruct((B, H, C, W), jnp.float32),
        scratch_shapes=[
            pltpu.VMEM((2 * B, C), jnp.float32),
        ],
    )(z, gw_t, bw_t,
      bn_gamma.reshape(1, C), bn_beta.reshape(1, C),
      bn_mean.reshape(1, C), bn_var.reshape(1, C), xt)

    return jnp.transpose(out_t, (0, 1, 3, 2))
